# baseline (speedup 1.0000x reference)
import functools

import jax
import jax.numpy as jnp
from jax.experimental import pallas as pl
from jax.experimental.pallas import tpu as pltpu

H = 32
R = 32
TOPK = 16
LN_EPS = 1e-5
NEG = -1e30
PACK = 4
ROW_TILE4 = 1024

(I_BQ, I_GQ, I_BETQ,
 I_BR1, I_GR1, I_BETR1, I_BR2,
 I_BF1, I_GF1, I_BETF1, I_BF2,
 I_BO, I_GLN, I_BLN) = range(14)
NUM_VECS = 14

HP = H * PACK


def _round_up(n, m):
    return (n + m - 1) // m * m


def _erf(x):
    a1, a2, a3, a4, a5 = 0.254829592, -0.284496736, 1.421413741, -1.453152027, 1.061405429
    p = 0.3275911
    ax = jnp.abs(x)
    t = 1.0 / (1.0 + p * ax)
    poly = ((((a5 * t + a4) * t + a3) * t + a2) * t + a1) * t
    y = 1.0 - poly * jnp.exp(-ax * ax)
    return jnp.where(x >= 0.0, y, -y)


def _gelu(x):
    return 0.5 * x * (1.0 + _erf(x * 0.7071067811865476))


def _layer_norm(x, gamma, beta):
    mu = jnp.mean(x, axis=-1, keepdims=True)
    var = jnp.mean((x - mu) ** 2, axis=-1, keepdims=True)
    return (x - mu) * jax.lax.rsqrt(var + LN_EPS) * gamma + beta


def _hi_lo_sq_cat(y):
    yh = y.astype(jnp.bfloat16)
    yl = (y - yh.astype(jnp.float32)).astype(jnp.bfloat16)
    y2 = y * y
    y2h = y2.astype(jnp.bfloat16)
    y2l = (y2 - y2h.astype(jnp.float32)).astype(jnp.bfloat16)
    return jnp.concatenate([yh, yl, y2h, y2l], axis=1)


def _seg_ln(y, gamma4, beta4, am):
    mus = jnp.dot(_hi_lo_sq_cat(y), am, preferred_element_type=jnp.float32)
    mu = mus[:, 0:HP]
    var = mus[:, HP:2 * HP] - mu * mu
    return (y - mu) * jax.lax.rsqrt(var + LN_EPS) * gamma4 + beta4


def _rag_kernel(x_ref, k4_ref, wx4_ref, wr24_ref, kew4_ref, wf24_ref,
                am_ref, vecs4_ref, out_ref, ts_ref, kw_ref, *, topk, kk):
    x = x_ref[...]
    vecs = vecs4_ref[...]
    am = am_ref[...]

    def v(i):
        return vecs[i:i + 1, :]

    xw = jnp.dot(x, wx4_ref[...], preferred_element_type=jnp.float32)
    q = _seg_ln(xw[:, 0:HP] + v(I_BQ), v(I_GQ), v(I_BETQ), am)
    rh = _gelu(_seg_ln(xw[:, HP:2 * HP] + v(I_BR1), v(I_GR1), v(I_BETR1), am))
    r = jnp.dot(rh, wr24_ref[...], preferred_element_type=jnp.float32) + v(I_BR2)
    qs = q + r
    f1a = xw[:, 2 * HP:3 * HP]

    s4 = jnp.dot(qs, k4_ref[...], preferred_element_type=jnp.float32)

    tm4 = s4.shape[0]
    tcol = jax.lax.broadcasted_iota(jnp.int32, (tm4, topk), 1)
    ts_parts, kw_parts, p_parts, rz_parts = [], [], [], []
    for j in range(PACK):
        s = s4[:, j * kk:(j + 1) * kk]
        ts = jnp.zeros((tm4, topk), jnp.float32)
        if kk == 4 * 128:
            def ce(a, b):
                return jnp.maximum(a, b), jnp.minimum(a, b)
            c0, c1, c2, c3 = (s[:, c * 128:(c + 1) * 128] for c in range(4))
            c0, c1 = ce(c0, c1)
            c2, c3 = ce(c2, c3)
            c0, c2 = ce(c0, c2)
            c1, c3 = ce(c1, c3)
            c1, c2 = ce(c1, c2)
            m0 = None
            for t in range(topk):
                m = jnp.max(c0, axis=-1, keepdims=True)
                if t == 0:
                    m0 = m
                ts = jnp.where(tcol == t, m, ts)
                hit = c0 == m
                c0 = jnp.where(hit, c1, c0)
                c1 = jnp.where(hit, c2, c1)
                c2 = jnp.where(hit, c3, c2)
                c3 = jnp.where(hit, NEG, c3)
        else:
            m = jnp.max(s, axis=-1, keepdims=True)
            m0 = m
            ts = jnp.where(tcol == 0, m, ts)
            for t in range(1, topk):
                m = jnp.max(jnp.where(s < m, s, NEG), axis=-1, keepdims=True)
                ts = jnp.where(tcol == t, m, ts)
        m15 = m
        e16 = jnp.exp(ts - m0)
        rz = 1.0 / jnp.sum(e16, axis=-1, keepdims=True)
        e = jnp.exp(s - m0)
        p_parts.append(jnp.where(s >= m15, e, 0.0).astype(jnp.bfloat16))
        ts_parts.append(ts)
        kw_parts.append(e16 * rz)
        rz_parts.append(rz)

    ts_ref[...] = jnp.concatenate(ts_parts, axis=1)
    kw_ref[...] = jnp.concatenate(kw_parts, axis=1)
    probs = jnp.concatenate(p_parts, axis=1)

    rw = jnp.dot(probs, kew4_ref[...], preferred_element_type=jnp.float32)
    lcol = jax.lax.broadcasted_iota(jnp.int32, (tm4, HP), 1)
    rzb = jnp.where(lcol < H, rz_parts[0],
                    jnp.where(lcol < 2 * H, rz_parts[1],
                              jnp.where(lcol < 3 * H, rz_parts[2], rz_parts[3])))
    rw = rw * jnp.concatenate([rzb, rzb], axis=1)
    f1b = rw[:, 0:HP]
    out_lin = rw[:, HP:2 * HP] + v(I_BO)

    h1 = _gelu(_seg_ln(f1a + f1b + v(I_BF1), v(I_GF1), v(I_BETF1), am))
    fw = jax.nn.sigmoid(
        jnp.dot(h1.astype(jnp.bfloat16), wf24_ref[...],
                preferred_element_type=jnp.float32) + v(I_BF2))
    out = x * (1.0 - fw) + out_lin * fw
    out_ref[...] = _seg_ln(out, v(I_GLN), v(I_BLN), am)


def _full(shape):
    n = len(shape)
    return pl.BlockSpec(shape, lambda i: (0,) * n)


@jax.jit
def kernel(hidden_states, knowledge_bank, wk, bk, gk, betk, wx, wr2, wrv, wf2, vecs):
    B, S, _ = hidden_states.shape
    K = knowledge_bank.shape[0]
    topk = min(TOPK, K)
    N = B * S
    x = hidden_states.reshape(N, H).astype(jnp.float32)
    kb = knowledge_bank.astype(jnp.float32)

    k_enc = _layer_norm(
        jnp.dot(kb, wk, preferred_element_type=jnp.float32) + bk, gk, betk)
    ket = k_enc.T
    kew = jnp.dot(k_enc, wrv, preferred_element_type=jnp.float32)

    eye4 = jnp.eye(PACK, dtype=jnp.float32)
    kr = lambda w: jnp.kron(eye4, w)
    wx4 = jnp.concatenate([kr(wx[:, 0:R]), kr(wx[:, R:2 * R]), kr(wx[:, 2 * R:3 * R])],
                          axis=1)
    wr24 = kr(wr2)
    k4 = kr(ket)
    kew4 = jnp.concatenate([kr(kew[:, 0:H]), kr(kew[:, H:2 * H])],
                           axis=1).astype(jnp.bfloat16)
    wf24 = kr(wf2).astype(jnp.bfloat16)
    vecs4 = jnp.tile(vecs, (1, PACK))
    a32 = kr(jnp.full((H, H), 1.0 / H, jnp.float32))
    z = jnp.zeros_like(a32)
    am = jnp.concatenate([
        jnp.concatenate([a32, z], axis=1),
        jnp.concatenate([a32, z], axis=1),
        jnp.concatenate([z, a32], axis=1),
        jnp.concatenate([z, a32], axis=1),
    ], axis=0).astype(jnp.bfloat16)

    n4 = pl.cdiv(N, PACK)
    tm4 = min(ROW_TILE4, max(8, _round_up(pl.cdiv(n4, 2), 8)))
    n4_pad = _round_up(n4, tm4)
    if n4_pad * PACK != N:
        x = jnp.pad(x, ((0, n4_pad * PACK - N), (0, 0)))
    x4 = x.reshape(n4_pad, HP)

    consts = (k4, wx4, wr24, kew4, wf24, am, vecs4)
    out4, ts4, kw4 = pl.pallas_call(
        functools.partial(_rag_kernel, topk=topk, kk=K),
        out_shape=(jax.ShapeDtypeStruct((n4_pad, HP), jnp.float32),
                   jax.ShapeDtypeStruct((n4_pad, PACK * topk), jnp.float32),
                   jax.ShapeDtypeStruct((n4_pad, PACK * topk), jnp.float32)),
        grid=(n4_pad // tm4,),
        in_specs=[pl.BlockSpec((tm4, HP), lambda i: (i, 0))]
                 + [_full(a.shape) for a in consts],
        out_specs=(pl.BlockSpec((tm4, HP), lambda i: (i, 0)),
                   pl.BlockSpec((tm4, PACK * topk), lambda i: (i, 0)),
                   pl.BlockSpec((tm4, PACK * topk), lambda i: (i, 0))),
        compiler_params=pltpu.CompilerParams(dimension_semantics=("parallel",)),
    )(x4, *consts)

    return {
        "hidden_states": out4.reshape(-1, H)[:N].reshape(B, S, H),
        "retrieval_scores": ts4.reshape(-1, topk)[:N].reshape(B, S, topk),
        "knowledge_weights": kw4.reshape(-1, topk)[:N].reshape(B, S, topk),
    }


# speedup vs baseline: 1.1841x; 1.1841x over previous
import functools

import jax
import jax.numpy as jnp
from jax.experimental import pallas as pl
from jax.experimental.pallas import tpu as pltpu

H = 32
R = 32
TOPK = 16
LN_EPS = 1e-5
NEG = -1e30
PACK = 4
ROW_TILE4 = 1024

(I_BQ, I_GQ, I_BETQ,
 I_BR1, I_GR1, I_BETR1, I_BR2,
 I_BF1, I_GF1, I_BETF1, I_BF2,
 I_BO, I_GLN, I_BLN) = range(14)
NUM_VECS = 14

HP = H * PACK


def _round_up(n, m):
    return (n + m - 1) // m * m


def _erf(x):
    a1, a2, a3, a4, a5 = 0.254829592, -0.284496736, 1.421413741, -1.453152027, 1.061405429
    p = 0.3275911
    ax = jnp.abs(x)
    t = 1.0 / (1.0 + p * ax)
    poly = ((((a5 * t + a4) * t + a3) * t + a2) * t + a1) * t
    y = 1.0 - poly * jnp.exp(-ax * ax)
    return jnp.where(x >= 0.0, y, -y)


def _gelu(x):
    return 0.5 * x * (1.0 + _erf(x * 0.7071067811865476))


def _layer_norm(x, gamma, beta):
    mu = jnp.mean(x, axis=-1, keepdims=True)
    var = jnp.mean((x - mu) ** 2, axis=-1, keepdims=True)
    return (x - mu) * jax.lax.rsqrt(var + LN_EPS) * gamma + beta


def _hi_lo_sq_cat(y):
    yh = y.astype(jnp.bfloat16)
    yl = (y - yh.astype(jnp.float32)).astype(jnp.bfloat16)
    y2 = y * y
    y2h = y2.astype(jnp.bfloat16)
    y2l = (y2 - y2h.astype(jnp.float32)).astype(jnp.bfloat16)
    return jnp.concatenate([yh, yl, y2h, y2l], axis=1)


def _seg_ln(y, gamma4, beta4, am):
    mus = jnp.dot(_hi_lo_sq_cat(y), am, preferred_element_type=jnp.float32)
    mu = mus[:, 0:HP]
    var = mus[:, HP:2 * HP] - mu * mu
    return (y - mu) * jax.lax.rsqrt(var + LN_EPS) * gamma4 + beta4


def _rag_kernel(x_ref, k4_ref, wx4_ref, wr24_ref, kew4_ref, wf24_ref,
                am_ref, vecs4_ref, out_ref, ts_ref, kw_ref, *, topk, kk):
    x = x_ref[...]
    vecs = vecs4_ref[...]
    am = am_ref[...]

    def v(i):
        return vecs[i:i + 1, :]

    xw = jnp.dot(x, wx4_ref[...], preferred_element_type=jnp.float32)
    q = _seg_ln(xw[:, 0:HP] + v(I_BQ), v(I_GQ), v(I_BETQ), am)
    rh = _gelu(_seg_ln(xw[:, HP:2 * HP] + v(I_BR1), v(I_GR1), v(I_BETR1), am))
    r = jnp.dot(rh, wr24_ref[...], preferred_element_type=jnp.float32) + v(I_BR2)
    qs = q + r
    f1a = xw[:, 2 * HP:3 * HP]

    s4 = jnp.dot(qs, k4_ref[...], preferred_element_type=jnp.float32)

    tm4 = s4.shape[0]
    tcol = jax.lax.broadcasted_iota(jnp.int32, (tm4, topk), 1)
    ts_parts, kw_parts, p_parts, rz_parts = [], [], [], []
    for j in range(PACK):
        s = s4[:, j * kk:(j + 1) * kk]
        ts = jnp.zeros((tm4, topk), jnp.float32)
        m = jnp.max(s, axis=-1, keepdims=True)
        m0 = m
        ts = jnp.where(tcol == 0, m, ts)
        for t in range(1, topk):
            m = jnp.max(jnp.where(s < m, s, NEG), axis=-1, keepdims=True)
            ts = jnp.where(tcol == t, m, ts)
        m15 = m
        e16 = jnp.exp(ts - m0)
        rz = 1.0 / jnp.sum(e16, axis=-1, keepdims=True)
        e = jnp.exp(s - m0)
        p_parts.append(jnp.where(s >= m15, e, 0.0).astype(jnp.bfloat16))
        ts_parts.append(ts)
        kw_parts.append(e16 * rz)
        rz_parts.append(rz)

    ts_ref[...] = jnp.concatenate(ts_parts, axis=1)
    kw_ref[...] = jnp.concatenate(kw_parts, axis=1)
    probs = jnp.concatenate(p_parts, axis=1)

    rw = jnp.dot(probs, kew4_ref[...], preferred_element_type=jnp.float32)
    lcol = jax.lax.broadcasted_iota(jnp.int32, (tm4, HP), 1)
    rzb = jnp.where(lcol < H, rz_parts[0],
                    jnp.where(lcol < 2 * H, rz_parts[1],
                              jnp.where(lcol < 3 * H, rz_parts[2], rz_parts[3])))
    rw = rw * jnp.concatenate([rzb, rzb], axis=1)
    f1b = rw[:, 0:HP]
    out_lin = rw[:, HP:2 * HP] + v(I_BO)

    h1 = _gelu(_seg_ln(f1a + f1b + v(I_BF1), v(I_GF1), v(I_BETF1), am))
    fw = jax.nn.sigmoid(
        jnp.dot(h1.astype(jnp.bfloat16), wf24_ref[...],
                preferred_element_type=jnp.float32) + v(I_BF2))
    out = x * (1.0 - fw) + out_lin * fw
    out_ref[...] = _seg_ln(out, v(I_GLN), v(I_BLN), am)


def _full(shape):
    n = len(shape)
    return pl.BlockSpec(shape, lambda i: (0,) * n)


@jax.jit
def kernel(hidden_states, knowledge_bank, wk, bk, gk, betk, wx, wr2, wrv, wf2, vecs):
    B, S, _ = hidden_states.shape
    K = knowledge_bank.shape[0]
    topk = min(TOPK, K)
    N = B * S
    x = hidden_states.reshape(N, H).astype(jnp.float32)
    kb = knowledge_bank.astype(jnp.float32)

    k_enc = _layer_norm(
        jnp.dot(kb, wk, preferred_element_type=jnp.float32) + bk, gk, betk)
    ket = k_enc.T
    kew = jnp.dot(k_enc, wrv, preferred_element_type=jnp.float32)

    eye4 = jnp.eye(PACK, dtype=jnp.float32)
    kr = lambda w: jnp.kron(eye4, w)
    wx4 = jnp.concatenate([kr(wx[:, 0:R]), kr(wx[:, R:2 * R]), kr(wx[:, 2 * R:3 * R])],
                          axis=1)
    wr24 = kr(wr2)
    k4 = kr(ket)
    kew4 = jnp.concatenate([kr(kew[:, 0:H]), kr(kew[:, H:2 * H])],
                           axis=1).astype(jnp.bfloat16)
    wf24 = kr(wf2).astype(jnp.bfloat16)
    vecs4 = jnp.tile(vecs, (1, PACK))
    a32 = kr(jnp.full((H, H), 1.0 / H, jnp.float32))
    z = jnp.zeros_like(a32)
    am = jnp.concatenate([
        jnp.concatenate([a32, z], axis=1),
        jnp.concatenate([a32, z], axis=1),
        jnp.concatenate([z, a32], axis=1),
        jnp.concatenate([z, a32], axis=1),
    ], axis=0).astype(jnp.bfloat16)

    n4 = pl.cdiv(N, PACK)
    tm4 = min(ROW_TILE4, max(8, _round_up(pl.cdiv(n4, 2), 8)))
    n4_pad = _round_up(n4, tm4)
    if n4_pad * PACK != N:
        x = jnp.pad(x, ((0, n4_pad * PACK - N), (0, 0)))
    x4 = x.reshape(n4_pad, HP)

    consts = (k4, wx4, wr24, kew4, wf24, am, vecs4)
    out4, ts4, kw4 = pl.pallas_call(
        functools.partial(_rag_kernel, topk=topk, kk=K),
        out_shape=(jax.ShapeDtypeStruct((n4_pad, HP), jnp.float32),
                   jax.ShapeDtypeStruct((n4_pad, PACK * topk), jnp.float32),
                   jax.ShapeDtypeStruct((n4_pad, PACK * topk), jnp.float32)),
        grid=(n4_pad // tm4,),
        in_specs=[pl.BlockSpec((tm4, HP), lambda i: (i, 0))]
                 + [_full(a.shape) for a in consts],
        out_specs=(pl.BlockSpec((tm4, HP), lambda i: (i, 0)),
                   pl.BlockSpec((tm4, PACK * topk), lambda i: (i, 0)),
                   pl.BlockSpec((tm4, PACK * topk), lambda i: (i, 0))),
        compiler_params=pltpu.CompilerParams(dimension_semantics=("parallel",)),
    )(x4, *consts)

    return {
        "hidden_states": out4.reshape(-1, H)[:N].reshape(B, S, H),
        "retrieval_scores": ts4.reshape(-1, topk)[:N].reshape(B, S, topk),
        "knowledge_weights": kw4.reshape(-1, topk)[:N].reshape(B, S, topk),
    }
